# baseline (speedup 1.0000x reference)
import functools

import jax
import jax.numpy as jnp
from jax import lax
from jax.experimental import pallas as pl
from jax.experimental.pallas import tpu as pltpu
from jax.experimental.pallas import tpu_sc as plsc

N = 10000
H = 128
NW = 32
FLUSH = 8192
CAP = FLUSH + 10000 + 16
CAPW = 160000 + CAP + 8
SPW = 5120
E0P = NW * SPW
BT = 128
ROWBLK = 4
NROWS_W = 313


def _b16(x):
    return x.astype(jnp.bfloat16)


def _b16f(x):
    return x.astype(jnp.bfloat16).astype(jnp.float32)


def _dotb(a, b):
    return jnp.dot(_b16(a), _b16(b), preferred_element_type=jnp.float32)


def _widx():
    c = lax.axis_index("c")
    s = lax.axis_index("s")
    return s * 2 + c


def _extract_body(jflat, cols_o, vals_o, rows_o, cnt_o,
                  buf, colbuf, valbuf, rowbuf, outv, sem):
    wid = _widx()
    lo = (wid * 625) // 2
    hi = ((wid + 1) * 625) // 2
    iota = lax.iota(jnp.int32, 16)

    def row_chunk(tb, carry):
        cursor, flushed = carry
        rb = lo + tb * ROWBLK
        rbc = jnp.minimum(rb, N - ROWBLK)
        qoff = rb - rbc
        pltpu.async_copy(jflat.at[pl.ds(pl.multiple_of(rbc * N, 8), ROWBLK * N)], buf, sem).wait()

        def one_row(q, cur):
            r = rb + q
            base = (q + qoff) * N
            rowv = jnp.zeros((16,), jnp.int32) + r

            def emit_vec(v, colbase, cu):
                m = v != 0.0
                cnt = plsc.all_reduce_population_count(m)[0]
                plsc.store_compressed(colbuf.at[pl.ds(cu, 16)],
                                      iota + colbase, mask=m)
                plsc.store_compressed(valbuf.at[pl.ds(cu, 16)], v, mask=m)
                plsc.store_compressed(rowbuf.at[pl.ds(cu, 16)], rowv, mask=m)
                return cu + cnt

            def grp_body(k, cur2):
                b0 = base + k * 64
                v0 = buf[pl.ds(b0, 16)]
                v1 = buf[pl.ds(b0 + 16, 16)]
                v2 = buf[pl.ds(b0 + 32, 16)]
                v3 = buf[pl.ds(b0 + 48, 16)]
                mx = jnp.maximum(jnp.maximum(v0, v1), jnp.maximum(v2, v3))
                anypc = plsc.all_reduce_population_count(mx != 0.0)[0]

                def emit(cu):
                    c0 = k * 64
                    cu = emit_vec(v0, c0, cu)
                    cu = emit_vec(v1, c0 + 16, cu)
                    cu = emit_vec(v2, c0 + 32, cu)
                    cu = emit_vec(v3, c0 + 48, cu)
                    return cu

                return lax.cond(anypc > 0, emit, lambda cu: cu, cur2)

            def scan_row(cu):
                cu = lax.fori_loop(0, N // 64, grp_body, cu)
                return emit_vec(buf[pl.ds(base + (N // 64) * 64, 16)],
                                (N // 64) * 64, cu)

            return lax.cond(r < hi, scan_row, lambda cu: cu, cur)

        for q in range(ROWBLK):
            cursor = one_row(q, cursor)

        def do_flush(cf):
            cu, fl = cf
            pltpu.sync_copy(colbuf.at[pl.ds(0, FLUSH)],
                            cols_o.at[pl.ds(pl.multiple_of(wid * CAPW + fl, 8), FLUSH)])
            pltpu.sync_copy(valbuf.at[pl.ds(0, FLUSH)],
                            vals_o.at[pl.ds(pl.multiple_of(wid * CAPW + fl, 8), FLUSH)])
            pltpu.sync_copy(rowbuf.at[pl.ds(0, FLUSH)],
                            rows_o.at[pl.ds(pl.multiple_of(wid * CAPW + fl, 8), FLUSH)])
            nmove = (cu - FLUSH + 15) // 16

            def mv(j, _):
                colbuf[pl.ds(j * 16, 16)] = colbuf[pl.ds(FLUSH + j * 16, 16)]
                valbuf[pl.ds(j * 16, 16)] = valbuf[pl.ds(FLUSH + j * 16, 16)]
                rowbuf[pl.ds(j * 16, 16)] = rowbuf[pl.ds(FLUSH + j * 16, 16)]
                return 0

            lax.fori_loop(0, nmove, mv, 0)
            return cu - FLUSH, fl + FLUSH

        cursor, flushed = lax.cond(cursor >= FLUSH, do_flush,
                                   lambda cf: cf, (cursor, flushed))
        return cursor, flushed

    nblk = (NROWS_W + ROWBLK - 1) // ROWBLK
    cursor, flushed = lax.fori_loop(0, nblk, row_chunk, (0, 0))

    pltpu.sync_copy(colbuf, cols_o.at[pl.ds(pl.multiple_of(wid * CAPW + flushed, 8), CAP)])
    pltpu.sync_copy(valbuf, vals_o.at[pl.ds(pl.multiple_of(wid * CAPW + flushed, 8), CAP)])
    pltpu.sync_copy(rowbuf, rows_o.at[pl.ds(pl.multiple_of(wid * CAPW + flushed, 8), CAP)])
    total = flushed + cursor
    outv[...] = jnp.where(iota == 0, total, 0)
    pltpu.sync_copy(outv, cnt_o.at[pl.ds(pl.multiple_of(wid * 16, 8), 16)])


def _k1_extract(jflat):
    mesh = plsc.VectorSubcoreMesh(core_axis_name="c", subcore_axis_name="s")
    out_type = (
        jax.ShapeDtypeStruct((NW * CAPW,), jnp.int32),
        jax.ShapeDtypeStruct((NW * CAPW,), jnp.float32),
        jax.ShapeDtypeStruct((NW * CAPW,), jnp.int32),
        jax.ShapeDtypeStruct((NW * 16,), jnp.int32),
    )
    scratch = [
        pltpu.VMEM((ROWBLK * N,), jnp.float32),
        pltpu.VMEM((CAP,), jnp.int32),
        pltpu.VMEM((CAP,), jnp.float32),
        pltpu.VMEM((CAP,), jnp.int32),
        pltpu.VMEM((16,), jnp.int32),
        pltpu.SemaphoreType.DMA,
    ]
    return pl.kernel(_extract_body, out_type=out_type, mesh=mesh,
                     scratch_types=scratch,
                     compiler_params=pltpu.CompilerParams(
                         needs_layout_passes=False))(jflat)


def _gather_body(colsf, valsf, rowsf, cnt_i, mvar,
                 agg1_o, agg2_o, vals_o, rows_o, nnz_o,
                 cvm, offs, sidx, ridx, cidx, vstage, mrow, mcol, nstage, sem):
    wid = _widx()
    iota = lax.iota(jnp.int32, 16)
    zeros16 = jnp.zeros((16,), jnp.int32)
    pltpu.sync_copy(cnt_i, cvm)
    c_lo = plsc.load_gather(cvm, [iota * 16])
    c_hi = plsc.load_gather(cvm, [(iota + 16) * 16])
    inc_lo = plsc.cumsum(c_lo)
    tot_lo = jnp.max(inc_lo)
    inc_hi = plsc.cumsum(c_hi)
    nnz = tot_lo + jnp.max(inc_hi)
    exc_lo = inc_lo - c_lo
    exc_hi = inc_hi - c_hi + tot_lo
    offs[pl.ds(0, 16)] = exc_lo
    offs[pl.ds(16, 16)] = exc_hi
    offs_sc = [exc_lo[k] for k in range(16)] + [exc_hi[k] for k in range(16)]

    @pl.when(wid == 0)
    def _():
        nstage[...] = jnp.where(iota == 0, nnz, 0)
        pltpu.sync_copy(nstage, nnz_o)

    def chunk(t, _):
        g0 = wid * SPW + t * BT
        for u in range(BT // 16):
            gvec = g0 + u * 16 + iota
            wv = zeros16 - 1
            for osc in offs_sc:
                wv = wv + (gvec >= osc).astype(jnp.int32)
            srcpos = gvec - plsc.load_gather(offs, [wv])
            srcflat = wv * CAPW + srcpos
            srcflat = jnp.where(gvec < nnz, srcflat, 0)
            sidx[0, pl.ds(u * 16, 16)] = srcflat
        pltpu.async_copy(colsf.at[sidx.at[0]], cidx.at[0], sem).wait()
        pltpu.async_copy(rowsf.at[sidx.at[0]], ridx.at[0], sem).wait()
        for u in range(BT // 16):
            cv = cidx[0, pl.ds(u * 16, 16)]
            cidx[0, pl.ds(u * 16, 16)] = jnp.clip(cv, 0, N - 1)
            rv = ridx[0, pl.ds(u * 16, 16)]
            ridx[0, pl.ds(u * 16, 16)] = jnp.clip(rv, 0, N - 1)
        pltpu.async_copy(valsf.at[sidx.at[0]], vstage, sem).wait()
        pltpu.sync_copy(vstage, vals_o.at[pl.ds(pl.multiple_of(g0, 8), BT)])
        pltpu.sync_copy(ridx.at[0], rows_o.at[pl.ds(pl.multiple_of(g0, 8), BT)])
        pltpu.async_copy(mvar.at[ridx.at[0]], mrow, sem).wait()
        pltpu.sync_copy(mrow, agg1_o.at[pl.ds(g0, BT), :])
        pltpu.async_copy(mvar.at[cidx.at[0]], mcol, sem).wait()
        pltpu.sync_copy(mcol, agg2_o.at[pl.ds(g0, BT), :])
        return 0

    lax.fori_loop(0, SPW // BT, chunk, 0)


def _k4_gather(colsf, valsf, rowsf, cnts, mvar):
    mesh = plsc.VectorSubcoreMesh(core_axis_name="c", subcore_axis_name="s")
    out_type = (
        jax.ShapeDtypeStruct((E0P, H), jnp.float32),
        jax.ShapeDtypeStruct((E0P, H), jnp.float32),
        jax.ShapeDtypeStruct((E0P,), jnp.float32),
        jax.ShapeDtypeStruct((E0P,), jnp.int32),
        jax.ShapeDtypeStruct((16,), jnp.int32),
    )
    scratch = [
        pltpu.VMEM((NW * 16,), jnp.int32),
        pltpu.VMEM((NW,), jnp.int32),
        pltpu.VMEM((1, BT), jnp.int32),
        pltpu.VMEM((1, BT), jnp.int32),
        pltpu.VMEM((1, BT), jnp.int32),
        pltpu.VMEM((BT,), jnp.float32),
        pltpu.VMEM((BT, H), jnp.float32),
        pltpu.VMEM((BT, H), jnp.float32),
        pltpu.VMEM((16,), jnp.int32),
        pltpu.SemaphoreType.DMA,
    ]
    return pl.kernel(_gather_body, out_type=out_type, mesh=mesh,
                     scratch_types=scratch,
                     compiler_params=pltpu.CompilerParams(
                         needs_layout_passes=False))(colsf, valsf, rowsf, cnts, mvar)


def _scatter_body(msg, rowsE, zrows, nm_o, acc, mstage, ridx):
    c = lax.axis_index("c")
    s = lax.axis_index("s")
    r0 = s * 624
    pltpu.sync_copy(zrows.at[pl.ds(r0, 624), :], acc.at[pl.ds(r0, 624), :])

    @pl.when(s == 0)
    def _():
        pltpu.sync_copy(zrows.at[pl.ds(9984, 16), :], acc.at[pl.ds(9984, 16), :])

    plsc.subcore_barrier()
    half = E0P // 2

    def chunk(t, _):
        base = c * half + s * SPW + t * BT
        pltpu.sync_copy(msg.at[pl.ds(base, BT), :], mstage)
        pltpu.sync_copy(rowsE.at[pl.ds(pl.multiple_of(base, 8), BT)], ridx.at[0])
        pltpu.sync_copy(mstage, acc.at[ridx.at[0]], add=True)
        return 0

    lax.fori_loop(0, SPW // BT, chunk, 0)
    plsc.subcore_barrier()
    pltpu.sync_copy(acc.at[pl.ds(r0, 624), :], nm_o.at[c, pl.ds(r0, 624), :])

    @pl.when(s == 0)
    def _():
        pltpu.sync_copy(acc.at[pl.ds(9984, 16), :], nm_o.at[c, pl.ds(9984, 16), :])


def _k6_scatter(msg, rowsE, zrows):
    mesh = plsc.VectorSubcoreMesh(core_axis_name="c", subcore_axis_name="s")
    out_type = jax.ShapeDtypeStruct((2, N, H), jnp.float32)
    scratch = [
        pltpu.VMEM_SHARED((N, H), jnp.float32),
        pltpu.VMEM((BT, H), jnp.float32),
        pltpu.VMEM((1, BT), jnp.int32),
    ]
    return pl.kernel(_scatter_body, out_type=out_type, mesh=mesh,
                     scratch_types=scratch,
                     compiler_params=pltpu.CompilerParams(
                         needs_layout_passes=False))(msg, rowsE, zrows)


def _stats_body(j_ref, rc_ref, rs_ref, cc_ref, cs_ref):
    x = j_ref[...]
    xb = _b16f(x)
    m = (x != 0.0).astype(jnp.float32)
    rc_ref[...] = jnp.sum(m, axis=1, keepdims=True)
    rs_ref[...] = jnp.sum(xb, axis=1, keepdims=True)

    @pl.when(pl.program_id(0) == 0)
    def _():
        cc_ref[...] = jnp.zeros_like(cc_ref)
        cs_ref[...] = jnp.zeros_like(cs_ref)

    cc_ref[...] += jnp.sum(m, axis=0, keepdims=True)
    cs_ref[...] += jnp.sum(xb, axis=0, keepdims=True)


def _k0_stats(J):
    BR = 80
    grid = (N // BR,)
    return pl.pallas_call(
        _stats_body,
        grid=grid,
        in_specs=[pl.BlockSpec((BR, N), lambda i: (i, 0))],
        out_specs=[
            pl.BlockSpec((BR, 1), lambda i: (i, 0)),
            pl.BlockSpec((BR, 1), lambda i: (i, 0)),
            pl.BlockSpec((1, N), lambda i: (0, 0)),
            pl.BlockSpec((1, N), lambda i: (0, 0)),
        ],
        out_shape=[
            jax.ShapeDtypeStruct((N, 1), jnp.float32),
            jax.ShapeDtypeStruct((N, 1), jnp.float32),
            jax.ShapeDtypeStruct((1, N), jnp.float32),
            jax.ShapeDtypeStruct((1, N), jnp.float32),
        ],
    )(J)


def _node_body(rc, rs, cc, cs, c01, c1, wih, whh, bih, bhh, mv_ref):
    deg = rc[...] + cc[...]
    sv = rs[...] + cs[...]
    c0b = _b16f(c01[...])
    agg = deg * c0b[0:1, :] + sv * c0b[1:2, :]
    gi = _dotb(agg, wih[...]) + bih[...]
    hb = bhh[...]
    r = jax.nn.sigmoid(gi[:, 0:H] + hb[:, 0:H])
    z = jax.nn.sigmoid(gi[:, H:2 * H] + hb[:, H:2 * H])
    n = jnp.tanh(gi[:, 2 * H:] + r * hb[:, 2 * H:])
    h1 = (1.0 - z) * n
    mv_ref[...] = _dotb(h1, c1[...])


def _k3_node(rc, rs, ccT, csT, c01, c1, wih, whh, bih, bhh):
    BN = 200
    grid = (N // BN,)
    full = lambda shape: pl.BlockSpec(shape, lambda i: tuple(0 for _ in shape))
    return pl.pallas_call(
        _node_body,
        grid=grid,
        in_specs=[
            pl.BlockSpec((BN, 1), lambda i: (i, 0)),
            pl.BlockSpec((BN, 1), lambda i: (i, 0)),
            pl.BlockSpec((BN, 1), lambda i: (i, 0)),
            pl.BlockSpec((BN, 1), lambda i: (i, 0)),
            full((2, H)), full((H, H)), full((H, 3 * H)), full((H, 3 * H)),
            full((1, 3 * H)), full((1, 3 * H)),
        ],
        out_specs=pl.BlockSpec((BN, H), lambda i: (i, 0)),
        out_shape=jax.ShapeDtypeStruct((N, H), jnp.float32),
    )(rc, rs, ccT, csT, c01, c1, wih, whh, bih, bhh)


def _edge_body(a1, a2, vals, nnz_ref, wih, whh, bih, bhh,
               w1, b1, w2, b2, w3, b3, msg_ref, *, TE):
    nnz = nnz_ref[0]
    vv = vals[...]
    bi = bih[...]
    bh = bhh[...]
    whh_ = whh[...]
    whb = _b16f(whh_)
    ghf = whb[0:1, :] + _b16f(vv) * whb[1:2, :] + bh
    r2 = jax.nn.sigmoid(bi[:, 0:H] + ghf[:, 0:H])
    z2 = jax.nn.sigmoid(bi[:, H:2 * H] + ghf[:, H:2 * H])
    n2 = jnp.tanh(bi[:, 2 * H:] + r2 * ghf[:, 2 * H:])
    lane = lax.broadcasted_iota(jnp.int32, (TE, H), 1)
    h0 = jnp.where(lane == 0, 1.0, 0.0) + jnp.where(lane == 1, vv, 0.0)
    h1f = (1.0 - z2) * n2 + z2 * h0
    agg = a1[...] + a2[...]
    gi2 = _dotb(agg, wih[...]) + bi
    gh2 = _dotb(h1f, whh_) + bh
    r3 = jax.nn.sigmoid(gi2[:, 0:H] + gh2[:, 0:H])
    z3 = jax.nn.sigmoid(gi2[:, H:2 * H] + gh2[:, H:2 * H])
    n3 = jnp.tanh(gi2[:, 2 * H:] + r3 * gh2[:, 2 * H:])
    h2 = (1.0 - z3) * n3 + z3 * h1f
    msg = jax.nn.relu(_dotb(h2, w1[...]) + b1[...])
    msg = jax.nn.relu(_dotb(msg, w2[...]) + b2[...])
    msg = _dotb(msg, w3[...]) + b3[...]
    eidx = pl.program_id(0) * TE + lax.broadcasted_iota(jnp.int32, (TE, 1), 0)
    msg_ref[...] = jnp.where(eidx < nnz, msg, 0.0)


def _k5_edge(a1, a2, valsE, nnz11, wih, whh, bih, bhh, w1, b1, w2, b2, w3, b3):
    TE = 512
    grid = (E0P // TE,)
    full = lambda shape: pl.BlockSpec(shape, lambda i: tuple(0 for _ in shape))
    return pl.pallas_call(
        functools.partial(_edge_body, TE=TE),
        grid=grid,
        in_specs=[
            pl.BlockSpec((TE, H), lambda i: (i, 0)),
            pl.BlockSpec((TE, H), lambda i: (i, 0)),
            pl.BlockSpec((TE, 1), lambda i: (i, 0)),
            pl.BlockSpec(memory_space=pltpu.SMEM),
            full((H, 3 * H)), full((H, 3 * H)),
            full((1, 3 * H)), full((1, 3 * H)),
            full((H, H)), full((1, H)), full((H, H)), full((1, H)),
            full((H, H)), full((1, H)),
        ],
        out_specs=pl.BlockSpec((TE, H), lambda i: (i, 0)),
        out_shape=jax.ShapeDtypeStruct((E0P, H), jnp.float32),
    )(a1, a2, valsE, nnz11, wih, whh, bih, bhh, w1, b1, w2, b2, w3, b3)


def _readout_body(nm0, nm1, w1, b1, w2, b2, w3, b3, out_ref):
    nm = nm0[...] + nm1[...]
    o = jax.nn.relu(_dotb(nm, w1[...]) + b1[...])
    o = jax.nn.relu(_dotb(o, w2[...]) + b2[...])
    o = _dotb(o, w3[...]) + b3[...]
    m = jnp.max(o, axis=1, keepdims=True)
    e = jnp.exp(o - m)
    out_ref[...] = e / jnp.sum(e, axis=1, keepdims=True)


def _k7_readout(nm0, nm1, w1, b1, w2, b2, w3, b3):
    BN = 200
    grid = (N // BN,)
    full = lambda shape: pl.BlockSpec(shape, lambda i: tuple(0 for _ in shape))
    return pl.pallas_call(
        _readout_body,
        grid=grid,
        in_specs=[
            pl.BlockSpec((BN, H), lambda i: (i, 0)),
            pl.BlockSpec((BN, H), lambda i: (i, 0)),
            full((H, H)), full((1, H)), full((H, H)), full((1, H)),
            full((H, 2)), full((1, 2)),
        ],
        out_specs=pl.BlockSpec((BN, 2), lambda i: (i, 0)),
        out_shape=jax.ShapeDtypeStruct((N, 2), jnp.float32),
    )(nm0, nm1, w1, b1, w2, b2, w3, b3)


def kernel(J, b, conv_W, gru_W_ih, gru_W_hh, gru_b_ih, gru_b_hh,
           mp_W1, mp_b1, mp_W2, mp_b2, mp_W3, mp_b3,
           ro_W1, ro_b1, ro_W2, ro_b2, ro_W3, ro_b3):
    bih = gru_b_ih.reshape(1, 3 * H)
    bhh = gru_b_hh.reshape(1, 3 * H)
    rc, rs, cc, cs = _k0_stats(J)
    mvar = _k3_node(rc, rs, cc.reshape(N, 1), cs.reshape(N, 1),
                    conv_W[0][0:2], conv_W[1], gru_W_ih, gru_W_hh, bih, bhh)
    colsF, valsF, rowsF, cnts = _k1_extract(J.reshape(-1))
    agg1, agg2, valsE, rowsE, nnzv = _k4_gather(colsF, valsF, rowsF, cnts, mvar)
    nnz11 = nnzv[0:1]
    msg = _k5_edge(agg1, agg2, valsE.reshape(E0P, 1), nnz11,
                   gru_W_ih, gru_W_hh, bih, bhh,
                   mp_W1, mp_b1.reshape(1, H), mp_W2, mp_b2.reshape(1, H),
                   mp_W3, mp_b3.reshape(1, H))
    zrows = jnp.zeros((N, H), jnp.float32)
    nm2 = _k6_scatter(msg, rowsE, zrows)
    out = _k7_readout(nm2[0], nm2[1], ro_W1, ro_b1.reshape(1, H),
                      ro_W2, ro_b2.reshape(1, H), ro_W3, ro_b3.reshape(1, 2))
    return out


# speedup vs baseline: 1.4876x; 1.4876x over previous
import functools

import jax
import jax.numpy as jnp
from jax import lax
from jax.experimental import pallas as pl
from jax.experimental.pallas import tpu as pltpu
from jax.experimental.pallas import tpu_sc as plsc

N = 10000
H = 128
NW = 32
FLUSH = 8192
CAP = FLUSH + 10000 + 16
CAPW = 160000 + CAP + 8
SPW = 5120
E0P = NW * SPW
BT = 128
ROWBLK = 4
NROWS_W = 313


def _b16(x):
    return x.astype(jnp.bfloat16)


def _b16f(x):
    return x.astype(jnp.bfloat16).astype(jnp.float32)


def _dotb(a, b):
    return jnp.dot(_b16(a), _b16(b), preferred_element_type=jnp.float32)


def _widx():
    c = lax.axis_index("c")
    s = lax.axis_index("s")
    return s * 2 + c


def _extract_body(jflat, cols_o, vals_o, rows_o, cnt_o,
                  buf, colbuf, valbuf, rowbuf, outv, sem):
    wid = _widx()
    lo = (wid * 625) // 2
    hi = ((wid + 1) * 625) // 2
    iota = lax.iota(jnp.int32, 16)

    def row_chunk(tb, carry):
        cursor, flushed = carry
        rb = lo + tb * ROWBLK
        rbc = jnp.minimum(rb, N - ROWBLK)
        qoff = rb - rbc
        pltpu.async_copy(jflat.at[pl.ds(pl.multiple_of(rbc * N, 8), ROWBLK * N)], buf, sem).wait()

        def one_row(q, cur):
            r = rb + q
            base = (q + qoff) * N
            rowv = jnp.zeros((16,), jnp.int32) + r

            def emit_vec(v, colbase, cu):
                m = v != 0.0
                cnt = plsc.all_reduce_population_count(m)[0]
                plsc.store_compressed(colbuf.at[pl.ds(cu, 16)],
                                      iota + colbase, mask=m)
                plsc.store_compressed(valbuf.at[pl.ds(cu, 16)], v, mask=m)
                plsc.store_compressed(rowbuf.at[pl.ds(cu, 16)], rowv, mask=m)
                return cu + cnt

            def grp_body(k, cur2):
                b0 = base + k * 128
                vs = [buf[pl.ds(b0 + 16 * j, 16)] for j in range(8)]
                m01 = jnp.maximum(vs[0], vs[1])
                m23 = jnp.maximum(vs[2], vs[3])
                m45 = jnp.maximum(vs[4], vs[5])
                m67 = jnp.maximum(vs[6], vs[7])
                mx = jnp.maximum(jnp.maximum(m01, m23), jnp.maximum(m45, m67))
                anypc = plsc.all_reduce_population_count(mx != 0.0)[0]

                def emit(cu):
                    c0 = k * 128
                    for j in range(8):
                        cu = emit_vec(vs[j], c0 + 16 * j, cu)
                    return cu

                return lax.cond(anypc > 0, emit, lambda cu: cu, cur2)

            def scan_row(cu):
                cu = lax.fori_loop(0, N // 128, grp_body, cu)
                cu = emit_vec(buf[pl.ds(base + (N // 128) * 128, 16)],
                              (N // 128) * 128, cu)
                return cu

            return lax.cond(r < hi, scan_row, lambda cu: cu, cur)

        for q in range(ROWBLK):
            cursor = one_row(q, cursor)

        def do_flush(cf):
            cu, fl = cf
            pltpu.sync_copy(colbuf.at[pl.ds(0, FLUSH)],
                            cols_o.at[pl.ds(pl.multiple_of(wid * CAPW + fl, 8), FLUSH)])
            pltpu.sync_copy(valbuf.at[pl.ds(0, FLUSH)],
                            vals_o.at[pl.ds(pl.multiple_of(wid * CAPW + fl, 8), FLUSH)])
            pltpu.sync_copy(rowbuf.at[pl.ds(0, FLUSH)],
                            rows_o.at[pl.ds(pl.multiple_of(wid * CAPW + fl, 8), FLUSH)])
            nmove = (cu - FLUSH + 15) // 16

            def mv(j, _):
                colbuf[pl.ds(j * 16, 16)] = colbuf[pl.ds(FLUSH + j * 16, 16)]
                valbuf[pl.ds(j * 16, 16)] = valbuf[pl.ds(FLUSH + j * 16, 16)]
                rowbuf[pl.ds(j * 16, 16)] = rowbuf[pl.ds(FLUSH + j * 16, 16)]
                return 0

            lax.fori_loop(0, nmove, mv, 0)
            return cu - FLUSH, fl + FLUSH

        cursor, flushed = lax.cond(cursor >= FLUSH, do_flush,
                                   lambda cf: cf, (cursor, flushed))
        return cursor, flushed

    nblk = (NROWS_W + ROWBLK - 1) // ROWBLK
    cursor, flushed = lax.fori_loop(0, nblk, row_chunk, (0, 0))

    pltpu.sync_copy(colbuf, cols_o.at[pl.ds(pl.multiple_of(wid * CAPW + flushed, 8), CAP)])
    pltpu.sync_copy(valbuf, vals_o.at[pl.ds(pl.multiple_of(wid * CAPW + flushed, 8), CAP)])
    pltpu.sync_copy(rowbuf, rows_o.at[pl.ds(pl.multiple_of(wid * CAPW + flushed, 8), CAP)])
    total = flushed + cursor
    outv[...] = jnp.where(iota == 0, total, 0)
    pltpu.sync_copy(outv, cnt_o.at[pl.ds(pl.multiple_of(wid * 16, 8), 16)])


def _k1_extract(jflat):
    mesh = plsc.VectorSubcoreMesh(core_axis_name="c", subcore_axis_name="s")
    out_type = (
        jax.ShapeDtypeStruct((NW * CAPW,), jnp.int32),
        jax.ShapeDtypeStruct((NW * CAPW,), jnp.float32),
        jax.ShapeDtypeStruct((NW * CAPW,), jnp.int32),
        jax.ShapeDtypeStruct((NW * 16,), jnp.int32),
    )
    scratch = [
        pltpu.VMEM((ROWBLK * N,), jnp.float32),
        pltpu.VMEM((CAP,), jnp.int32),
        pltpu.VMEM((CAP,), jnp.float32),
        pltpu.VMEM((CAP,), jnp.int32),
        pltpu.VMEM((16,), jnp.int32),
        pltpu.SemaphoreType.DMA,
    ]
    return pl.kernel(_extract_body, out_type=out_type, mesh=mesh,
                     scratch_types=scratch,
                     compiler_params=pltpu.CompilerParams(
                         needs_layout_passes=False))(jflat)


def _gather_body(colsf, valsf, rowsf, cnt_i, mvar,
                 agg1_o, agg2_o, vals_o, rows_o, nnz_o,
                 cvm, offs, sidx, ridx, cidx, vstage, mrow, mcol, nstage,
                 sem, sem2, sem3):
    wid = _widx()
    iota = lax.iota(jnp.int32, 16)
    zeros16 = jnp.zeros((16,), jnp.int32)
    pltpu.sync_copy(cnt_i, cvm)
    c_lo = plsc.load_gather(cvm, [iota * 16])
    c_hi = plsc.load_gather(cvm, [(iota + 16) * 16])
    inc_lo = plsc.cumsum(c_lo)
    tot_lo = jnp.max(inc_lo)
    inc_hi = plsc.cumsum(c_hi)
    nnz = tot_lo + jnp.max(inc_hi)
    exc_lo = inc_lo - c_lo
    exc_hi = inc_hi - c_hi + tot_lo
    offs[pl.ds(0, 16)] = exc_lo
    offs[pl.ds(16, 16)] = exc_hi
    offs_sc = [exc_lo[k] for k in range(16)] + [exc_hi[k] for k in range(16)]

    @pl.when(wid == 0)
    def _():
        nstage[...] = jnp.where(iota == 0, nnz, 0)
        pltpu.sync_copy(nstage, nnz_o)

    def chunk(t, _):
        g0 = wid * SPW + t * BT
        for u in range(BT // 16):
            gvec = g0 + u * 16 + iota
            wv = zeros16 - 1
            for osc in offs_sc:
                wv = wv + (gvec >= osc).astype(jnp.int32)
            srcpos = gvec - plsc.load_gather(offs, [wv])
            srcflat = wv * CAPW + srcpos
            srcflat = jnp.where(gvec < nnz, srcflat, 0)
            sidx[0, pl.ds(u * 16, 16)] = srcflat
        d1 = pltpu.async_copy(colsf.at[sidx.at[0]], cidx.at[0], sem)
        d2 = pltpu.async_copy(rowsf.at[sidx.at[0]], ridx.at[0], sem2)
        d3 = pltpu.async_copy(valsf.at[sidx.at[0]], vstage, sem3)
        d1.wait()
        d2.wait()
        d3.wait()
        for u in range(BT // 16):
            cv = cidx[0, pl.ds(u * 16, 16)]
            cidx[0, pl.ds(u * 16, 16)] = jnp.clip(cv, 0, N - 1)
            rv = ridx[0, pl.ds(u * 16, 16)]
            ridx[0, pl.ds(u * 16, 16)] = jnp.clip(rv, 0, N - 1)
        g1 = pltpu.async_copy(mvar.at[ridx.at[0]], mrow, sem)
        g2 = pltpu.async_copy(mvar.at[cidx.at[0]], mcol, sem2)
        pltpu.sync_copy(vstage, vals_o.at[pl.ds(pl.multiple_of(g0, 8), BT)])
        pltpu.sync_copy(ridx.at[0], rows_o.at[pl.ds(pl.multiple_of(g0, 8), BT)])
        g1.wait()
        pltpu.sync_copy(mrow, agg1_o.at[pl.ds(g0, BT), :])
        g2.wait()
        pltpu.sync_copy(mcol, agg2_o.at[pl.ds(g0, BT), :])
        return 0

    lax.fori_loop(0, SPW // BT, chunk, 0)


def _k4_gather(colsf, valsf, rowsf, cnts, mvar):
    mesh = plsc.VectorSubcoreMesh(core_axis_name="c", subcore_axis_name="s")
    out_type = (
        jax.ShapeDtypeStruct((E0P, H), jnp.float32),
        jax.ShapeDtypeStruct((E0P, H), jnp.float32),
        jax.ShapeDtypeStruct((E0P,), jnp.float32),
        jax.ShapeDtypeStruct((E0P,), jnp.int32),
        jax.ShapeDtypeStruct((16,), jnp.int32),
    )
    scratch = [
        pltpu.VMEM((NW * 16,), jnp.int32),
        pltpu.VMEM((NW,), jnp.int32),
        pltpu.VMEM((1, BT), jnp.int32),
        pltpu.VMEM((1, BT), jnp.int32),
        pltpu.VMEM((1, BT), jnp.int32),
        pltpu.VMEM((BT,), jnp.float32),
        pltpu.VMEM((BT, H), jnp.float32),
        pltpu.VMEM((BT, H), jnp.float32),
        pltpu.VMEM((16,), jnp.int32),
        pltpu.SemaphoreType.DMA,
        pltpu.SemaphoreType.DMA,
        pltpu.SemaphoreType.DMA,
    ]
    return pl.kernel(_gather_body, out_type=out_type, mesh=mesh,
                     scratch_types=scratch,
                     compiler_params=pltpu.CompilerParams(
                         needs_layout_passes=False))(colsf, valsf, rowsf, cnts, mvar)


def _scatter_body(msg, rowsE, zrows, nm_o, acc, mstage, ridx):
    c = lax.axis_index("c")
    s = lax.axis_index("s")
    r0 = s * 624
    pltpu.sync_copy(zrows.at[pl.ds(r0, 624), :], acc.at[pl.ds(r0, 624), :])

    @pl.when(s == 0)
    def _():
        pltpu.sync_copy(zrows.at[pl.ds(9984, 16), :], acc.at[pl.ds(9984, 16), :])

    plsc.subcore_barrier()
    half = E0P // 2

    def chunk(t, _):
        base = c * half + s * SPW + t * BT
        pltpu.sync_copy(msg.at[pl.ds(base, BT), :], mstage)
        pltpu.sync_copy(rowsE.at[pl.ds(pl.multiple_of(base, 8), BT)], ridx.at[0])
        pltpu.sync_copy(mstage, acc.at[ridx.at[0]], add=True)
        return 0

    lax.fori_loop(0, SPW // BT, chunk, 0)
    plsc.subcore_barrier()
    pltpu.sync_copy(acc.at[pl.ds(r0, 624), :], nm_o.at[c, pl.ds(r0, 624), :])

    @pl.when(s == 0)
    def _():
        pltpu.sync_copy(acc.at[pl.ds(9984, 16), :], nm_o.at[c, pl.ds(9984, 16), :])


def _k6_scatter(msg, rowsE, zrows):
    mesh = plsc.VectorSubcoreMesh(core_axis_name="c", subcore_axis_name="s")
    out_type = jax.ShapeDtypeStruct((2, N, H), jnp.float32)
    scratch = [
        pltpu.VMEM_SHARED((N, H), jnp.float32),
        pltpu.VMEM((BT, H), jnp.float32),
        pltpu.VMEM((1, BT), jnp.int32),
    ]
    return pl.kernel(_scatter_body, out_type=out_type, mesh=mesh,
                     scratch_types=scratch,
                     compiler_params=pltpu.CompilerParams(
                         needs_layout_passes=False))(msg, rowsE, zrows)


def _stats_body(j_ref, rc_ref, rs_ref, cc_ref, cs_ref):
    x = j_ref[...]
    xb = _b16f(x)
    m = (x != 0.0).astype(jnp.float32)
    rc_ref[...] = jnp.sum(m, axis=1, keepdims=True)
    rs_ref[...] = jnp.sum(xb, axis=1, keepdims=True)

    @pl.when(pl.program_id(0) == 0)
    def _():
        cc_ref[...] = jnp.zeros_like(cc_ref)
        cs_ref[...] = jnp.zeros_like(cs_ref)

    cc_ref[...] += jnp.sum(m, axis=0, keepdims=True)
    cs_ref[...] += jnp.sum(xb, axis=0, keepdims=True)


def _k0_stats(J):
    BR = 80
    grid = (N // BR,)
    return pl.pallas_call(
        _stats_body,
        grid=grid,
        in_specs=[pl.BlockSpec((BR, N), lambda i: (i, 0))],
        out_specs=[
            pl.BlockSpec((BR, 1), lambda i: (i, 0)),
            pl.BlockSpec((BR, 1), lambda i: (i, 0)),
            pl.BlockSpec((1, N), lambda i: (0, 0)),
            pl.BlockSpec((1, N), lambda i: (0, 0)),
        ],
        out_shape=[
            jax.ShapeDtypeStruct((N, 1), jnp.float32),
            jax.ShapeDtypeStruct((N, 1), jnp.float32),
            jax.ShapeDtypeStruct((1, N), jnp.float32),
            jax.ShapeDtypeStruct((1, N), jnp.float32),
        ],
    )(J)


def _node_body(rc, rs, cc, cs, c01, c1, wih, whh, bih, bhh, mv_ref):
    deg = rc[...] + cc[...]
    sv = rs[...] + cs[...]
    c0b = _b16f(c01[...])
    agg = deg * c0b[0:1, :] + sv * c0b[1:2, :]
    gi = _dotb(agg, wih[...]) + bih[...]
    hb = bhh[...]
    r = jax.nn.sigmoid(gi[:, 0:H] + hb[:, 0:H])
    z = jax.nn.sigmoid(gi[:, H:2 * H] + hb[:, H:2 * H])
    n = jnp.tanh(gi[:, 2 * H:] + r * hb[:, 2 * H:])
    h1 = (1.0 - z) * n
    mv_ref[...] = _dotb(h1, c1[...])


def _k3_node(rc, rs, ccT, csT, c01, c1, wih, whh, bih, bhh):
    BN = 200
    grid = (N // BN,)
    full = lambda shape: pl.BlockSpec(shape, lambda i: tuple(0 for _ in shape))
    return pl.pallas_call(
        _node_body,
        grid=grid,
        in_specs=[
            pl.BlockSpec((BN, 1), lambda i: (i, 0)),
            pl.BlockSpec((BN, 1), lambda i: (i, 0)),
            pl.BlockSpec((BN, 1), lambda i: (i, 0)),
            pl.BlockSpec((BN, 1), lambda i: (i, 0)),
            full((2, H)), full((H, H)), full((H, 3 * H)), full((H, 3 * H)),
            full((1, 3 * H)), full((1, 3 * H)),
        ],
        out_specs=pl.BlockSpec((BN, H), lambda i: (i, 0)),
        out_shape=jax.ShapeDtypeStruct((N, H), jnp.float32),
    )(rc, rs, ccT, csT, c01, c1, wih, whh, bih, bhh)


def _edge_body(a1, a2, vals, nnz_ref, wih, whh, bih, bhh,
               w1, b1, w2, b2, w3, b3, msg_ref, *, TE):
    nnz = nnz_ref[0]
    vv = vals[...]
    bi = bih[...]
    bh = bhh[...]
    whh_ = whh[...]
    whb = _b16f(whh_)
    ghf = whb[0:1, :] + _b16f(vv) * whb[1:2, :] + bh
    r2 = jax.nn.sigmoid(bi[:, 0:H] + ghf[:, 0:H])
    z2 = jax.nn.sigmoid(bi[:, H:2 * H] + ghf[:, H:2 * H])
    n2 = jnp.tanh(bi[:, 2 * H:] + r2 * ghf[:, 2 * H:])
    lane = lax.broadcasted_iota(jnp.int32, (TE, H), 1)
    h0 = jnp.where(lane == 0, 1.0, 0.0) + jnp.where(lane == 1, vv, 0.0)
    h1f = (1.0 - z2) * n2 + z2 * h0
    agg = a1[...] + a2[...]
    gi2 = _dotb(agg, wih[...]) + bi
    gh2 = _dotb(h1f, whh_) + bh
    r3 = jax.nn.sigmoid(gi2[:, 0:H] + gh2[:, 0:H])
    z3 = jax.nn.sigmoid(gi2[:, H:2 * H] + gh2[:, H:2 * H])
    n3 = jnp.tanh(gi2[:, 2 * H:] + r3 * gh2[:, 2 * H:])
    h2 = (1.0 - z3) * n3 + z3 * h1f
    msg = jax.nn.relu(_dotb(h2, w1[...]) + b1[...])
    msg = jax.nn.relu(_dotb(msg, w2[...]) + b2[...])
    msg = _dotb(msg, w3[...]) + b3[...]
    eidx = pl.program_id(0) * TE + lax.broadcasted_iota(jnp.int32, (TE, 1), 0)
    msg_ref[...] = jnp.where(eidx < nnz, msg, 0.0)


def _k5_edge(a1, a2, valsE, nnz11, wih, whh, bih, bhh, w1, b1, w2, b2, w3, b3):
    TE = 512
    grid = (E0P // TE,)
    full = lambda shape: pl.BlockSpec(shape, lambda i: tuple(0 for _ in shape))
    return pl.pallas_call(
        functools.partial(_edge_body, TE=TE),
        grid=grid,
        in_specs=[
            pl.BlockSpec((TE, H), lambda i: (i, 0)),
            pl.BlockSpec((TE, H), lambda i: (i, 0)),
            pl.BlockSpec((TE, 1), lambda i: (i, 0)),
            pl.BlockSpec(memory_space=pltpu.SMEM),
            full((H, 3 * H)), full((H, 3 * H)),
            full((1, 3 * H)), full((1, 3 * H)),
            full((H, H)), full((1, H)), full((H, H)), full((1, H)),
            full((H, H)), full((1, H)),
        ],
        out_specs=pl.BlockSpec((TE, H), lambda i: (i, 0)),
        out_shape=jax.ShapeDtypeStruct((E0P, H), jnp.float32),
    )(a1, a2, valsE, nnz11, wih, whh, bih, bhh, w1, b1, w2, b2, w3, b3)


def _readout_body(nm0, nm1, w1, b1, w2, b2, w3, b3, out_ref):
    nm = nm0[...] + nm1[...]
    o = jax.nn.relu(_dotb(nm, w1[...]) + b1[...])
    o = jax.nn.relu(_dotb(o, w2[...]) + b2[...])
    o = _dotb(o, w3[...]) + b3[...]
    m = jnp.max(o, axis=1, keepdims=True)
    e = jnp.exp(o - m)
    out_ref[...] = e / jnp.sum(e, axis=1, keepdims=True)


def _k7_readout(nm0, nm1, w1, b1, w2, b2, w3, b3):
    BN = 200
    grid = (N // BN,)
    full = lambda shape: pl.BlockSpec(shape, lambda i: tuple(0 for _ in shape))
    return pl.pallas_call(
        _readout_body,
        grid=grid,
        in_specs=[
            pl.BlockSpec((BN, H), lambda i: (i, 0)),
            pl.BlockSpec((BN, H), lambda i: (i, 0)),
            full((H, H)), full((1, H)), full((H, H)), full((1, H)),
            full((H, 2)), full((1, 2)),
        ],
        out_specs=pl.BlockSpec((BN, 2), lambda i: (i, 0)),
        out_shape=jax.ShapeDtypeStruct((N, 2), jnp.float32),
    )(nm0, nm1, w1, b1, w2, b2, w3, b3)


def kernel(J, b, conv_W, gru_W_ih, gru_W_hh, gru_b_ih, gru_b_hh,
           mp_W1, mp_b1, mp_W2, mp_b2, mp_W3, mp_b3,
           ro_W1, ro_b1, ro_W2, ro_b2, ro_W3, ro_b3):
    bih = gru_b_ih.reshape(1, 3 * H)
    bhh = gru_b_hh.reshape(1, 3 * H)
    rc, rs, cc, cs = _k0_stats(J)
    mvar = _k3_node(rc, rs, cc.reshape(N, 1), cs.reshape(N, 1),
                    conv_W[0][0:2], conv_W[1], gru_W_ih, gru_W_hh, bih, bhh)
    colsF, valsF, rowsF, cnts = _k1_extract(J.reshape(-1))
    agg1, agg2, valsE, rowsE, nnzv = _k4_gather(colsF, valsF, rowsF, cnts, mvar)
    nnz11 = nnzv[0:1]
    msg = _k5_edge(agg1, agg2, valsE.reshape(E0P, 1), nnz11,
                   gru_W_ih, gru_W_hh, bih, bhh,
                   mp_W1, mp_b1.reshape(1, H), mp_W2, mp_b2.reshape(1, H),
                   mp_W3, mp_b3.reshape(1, H))
    zrows = jnp.zeros((N, H), jnp.float32)
    nm2 = _k6_scatter(msg, rowsE, zrows)
    out = _k7_readout(nm2[0], nm2[1], ro_W1, ro_b1.reshape(1, H),
                      ro_W2, ro_b2.reshape(1, H), ro_W3, ro_b3.reshape(1, 2))
    return out


# speedup vs baseline: 1.6197x; 1.0888x over previous
import functools

import jax
import jax.numpy as jnp
from jax import lax
from jax.experimental import pallas as pl
from jax.experimental.pallas import tpu as pltpu
from jax.experimental.pallas import tpu_sc as plsc

N = 10000
H = 128
NW = 32
FLUSH = 8192
CAP = FLUSH + 10000 + 16
CAPW = 160000 + CAP + 8
SPW = 5120
E0P = NW * SPW
BT = 128
ROWBLK = 2
NROWS_W = 313


def _b16(x):
    return x.astype(jnp.bfloat16)


def _b16f(x):
    return x.astype(jnp.bfloat16).astype(jnp.float32)


def _dotb(a, b):
    return jnp.dot(_b16(a), _b16(b), preferred_element_type=jnp.float32)


def _widx():
    c = lax.axis_index("c")
    s = lax.axis_index("s")
    return s * 2 + c


def _extract_body(jflat, cols_o, vals_o, rows_o, cnt_o,
                  buf, buf2, colbuf, valbuf, rowbuf, outv, sem, sem2):
    wid = _widx()
    lo = (wid * 625) // 2
    hi = ((wid + 1) * 625) // 2
    iota = lax.iota(jnp.int32, 16)

    def fetch(blk, b, s):
        rb = lo + blk * ROWBLK
        rbc = jnp.minimum(rb, N - ROWBLK)
        pltpu.async_copy(
            jflat.at[pl.ds(pl.multiple_of(rbc * N, 8), ROWBLK * N)], b, s)
        return rb - rbc

    def proc_block(tb, buf, qoff, carry):
        cursor, flushed = carry
        rb = lo + tb * ROWBLK

        def one_row(q, cur):
            r = rb + q
            base = (q + qoff) * N
            rowv = jnp.zeros((16,), jnp.int32) + r

            def emit_vec(v, colbase, cu):
                m = v != 0.0
                cnt = plsc.all_reduce_population_count(m)[0]
                plsc.store_compressed(colbuf.at[pl.ds(cu, 16)],
                                      iota + colbase, mask=m)
                plsc.store_compressed(valbuf.at[pl.ds(cu, 16)], v, mask=m)
                plsc.store_compressed(rowbuf.at[pl.ds(cu, 16)], rowv, mask=m)
                return cu + cnt

            def grp_body(k, cur2):
                b0 = base + k * 128
                vs = [buf[pl.ds(b0 + 16 * j, 16)] for j in range(8)]
                m01 = jnp.maximum(vs[0], vs[1])
                m23 = jnp.maximum(vs[2], vs[3])
                m45 = jnp.maximum(vs[4], vs[5])
                m67 = jnp.maximum(vs[6], vs[7])
                mx = jnp.maximum(jnp.maximum(m01, m23), jnp.maximum(m45, m67))
                anypc = plsc.all_reduce_population_count(mx != 0.0)[0]

                def emit(cu):
                    c0 = k * 128
                    for j in range(8):
                        cu = emit_vec(vs[j], c0 + 16 * j, cu)
                    return cu

                return lax.cond(anypc > 0, emit, lambda cu: cu, cur2)

            def scan_row(cu):
                cu = lax.fori_loop(0, N // 128, grp_body, cu)
                cu = emit_vec(buf[pl.ds(base + (N // 128) * 128, 16)],
                              (N // 128) * 128, cu)
                return cu

            return lax.cond(r < hi, scan_row, lambda cu: cu, cur)

        for q in range(ROWBLK):
            cursor = one_row(q, cursor)

        def do_flush(cf):
            cu, fl = cf
            pltpu.sync_copy(colbuf.at[pl.ds(0, FLUSH)],
                            cols_o.at[pl.ds(pl.multiple_of(wid * CAPW + fl, 8), FLUSH)])
            pltpu.sync_copy(valbuf.at[pl.ds(0, FLUSH)],
                            vals_o.at[pl.ds(pl.multiple_of(wid * CAPW + fl, 8), FLUSH)])
            pltpu.sync_copy(rowbuf.at[pl.ds(0, FLUSH)],
                            rows_o.at[pl.ds(pl.multiple_of(wid * CAPW + fl, 8), FLUSH)])
            nmove = (cu - FLUSH + 15) // 16

            def mv(j, _):
                colbuf[pl.ds(j * 16, 16)] = colbuf[pl.ds(FLUSH + j * 16, 16)]
                valbuf[pl.ds(j * 16, 16)] = valbuf[pl.ds(FLUSH + j * 16, 16)]
                rowbuf[pl.ds(j * 16, 16)] = rowbuf[pl.ds(FLUSH + j * 16, 16)]
                return 0

            lax.fori_loop(0, nmove, mv, 0)
            return cu - FLUSH, fl + FLUSH

        cursor, flushed = lax.cond(cursor >= FLUSH, do_flush,
                                   lambda cf: cf, (cursor, flushed))
        return cursor, flushed

    npair = (NROWS_W + 2 * ROWBLK - 1) // (2 * ROWBLK)
    qoff0 = fetch(0, buf, sem)

    def pair_body(t, carry):
        cursor, flushed, qa = carry
        qb = fetch(2 * t + 1, buf2, sem2)
        pltpu.make_async_copy(
            jflat.at[pl.ds(0, ROWBLK * N)], buf, sem).wait()
        cursor, flushed = proc_block(2 * t, buf, qa, (cursor, flushed))
        qa2 = fetch(jnp.minimum(2 * t + 2, 2 * npair - 1), buf, sem)
        pltpu.make_async_copy(
            jflat.at[pl.ds(0, ROWBLK * N)], buf2, sem2).wait()
        cursor, flushed = proc_block(2 * t + 1, buf2, qb, (cursor, flushed))
        return cursor, flushed, qa2

    cursor, flushed, _ = lax.fori_loop(0, npair, pair_body, (0, 0, qoff0))
    pltpu.make_async_copy(jflat.at[pl.ds(0, ROWBLK * N)], buf, sem).wait()

    pltpu.sync_copy(colbuf, cols_o.at[pl.ds(pl.multiple_of(wid * CAPW + flushed, 8), CAP)])
    pltpu.sync_copy(valbuf, vals_o.at[pl.ds(pl.multiple_of(wid * CAPW + flushed, 8), CAP)])
    pltpu.sync_copy(rowbuf, rows_o.at[pl.ds(pl.multiple_of(wid * CAPW + flushed, 8), CAP)])
    total = flushed + cursor
    outv[...] = jnp.where(iota == 0, total, 0)
    pltpu.sync_copy(outv, cnt_o.at[pl.ds(pl.multiple_of(wid * 16, 8), 16)])


def _k1_extract(jflat):
    mesh = plsc.VectorSubcoreMesh(core_axis_name="c", subcore_axis_name="s")
    out_type = (
        jax.ShapeDtypeStruct((NW * CAPW,), jnp.int32),
        jax.ShapeDtypeStruct((NW * CAPW,), jnp.float32),
        jax.ShapeDtypeStruct((NW * CAPW,), jnp.int32),
        jax.ShapeDtypeStruct((NW * 16,), jnp.int32),
    )
    scratch = [
        pltpu.VMEM((ROWBLK * N,), jnp.float32),
        pltpu.VMEM((ROWBLK * N,), jnp.float32),
        pltpu.VMEM((CAP,), jnp.int32),
        pltpu.VMEM((CAP,), jnp.float32),
        pltpu.VMEM((CAP,), jnp.int32),
        pltpu.VMEM((16,), jnp.int32),
        pltpu.SemaphoreType.DMA,
        pltpu.SemaphoreType.DMA,
    ]
    return pl.kernel(_extract_body, out_type=out_type, mesh=mesh,
                     scratch_types=scratch,
                     compiler_params=pltpu.CompilerParams(
                         needs_layout_passes=False))(jflat)


def _gather_body(colsf, valsf, rowsf, cnt_i, mvar,
                 agg1_o, agg2_o, vals_o, rows_o, nnz_o,
                 cvm, offs, sidx, ridx, cidx, vstage, mrow, mcol, nstage,
                 sem, sem2, sem3):
    wid = _widx()
    iota = lax.iota(jnp.int32, 16)
    zeros16 = jnp.zeros((16,), jnp.int32)
    pltpu.sync_copy(cnt_i, cvm)
    c_lo = plsc.load_gather(cvm, [iota * 16])
    c_hi = plsc.load_gather(cvm, [(iota + 16) * 16])
    inc_lo = plsc.cumsum(c_lo)
    tot_lo = jnp.max(inc_lo)
    inc_hi = plsc.cumsum(c_hi)
    nnz = tot_lo + jnp.max(inc_hi)
    exc_lo = inc_lo - c_lo
    exc_hi = inc_hi - c_hi + tot_lo
    offs[pl.ds(0, 16)] = exc_lo
    offs[pl.ds(16, 16)] = exc_hi
    offs_sc = [exc_lo[k] for k in range(16)] + [exc_hi[k] for k in range(16)]

    @pl.when(wid == 0)
    def _():
        nstage[...] = jnp.where(iota == 0, nnz, 0)
        pltpu.sync_copy(nstage, nnz_o)

    def chunk(t, _):
        g0 = wid * SPW + t * BT
        for u in range(BT // 16):
            gvec = g0 + u * 16 + iota
            wv = zeros16 - 1
            for osc in offs_sc:
                wv = wv + (gvec >= osc).astype(jnp.int32)
            srcpos = gvec - plsc.load_gather(offs, [wv])
            srcflat = wv * CAPW + srcpos
            srcflat = jnp.where(gvec < nnz, srcflat, 0)
            sidx[0, pl.ds(u * 16, 16)] = srcflat
        d1 = pltpu.async_copy(colsf.at[sidx.at[0]], cidx.at[0], sem)
        d2 = pltpu.async_copy(rowsf.at[sidx.at[0]], ridx.at[0], sem2)
        d3 = pltpu.async_copy(valsf.at[sidx.at[0]], vstage, sem3)
        d1.wait()
        d2.wait()
        d3.wait()
        for u in range(BT // 16):
            cv = cidx[0, pl.ds(u * 16, 16)]
            cidx[0, pl.ds(u * 16, 16)] = jnp.clip(cv, 0, N - 1)
            rv = ridx[0, pl.ds(u * 16, 16)]
            ridx[0, pl.ds(u * 16, 16)] = jnp.clip(rv, 0, N - 1)
        g1 = pltpu.async_copy(mvar.at[ridx.at[0]], mrow, sem)
        g2 = pltpu.async_copy(mvar.at[cidx.at[0]], mcol, sem2)
        pltpu.sync_copy(vstage, vals_o.at[pl.ds(pl.multiple_of(g0, 8), BT)])
        pltpu.sync_copy(ridx.at[0], rows_o.at[pl.ds(pl.multiple_of(g0, 8), BT)])
        g1.wait()
        pltpu.sync_copy(mrow, agg1_o.at[pl.ds(g0, BT), :])
        g2.wait()
        pltpu.sync_copy(mcol, agg2_o.at[pl.ds(g0, BT), :])
        return 0

    lax.fori_loop(0, SPW // BT, chunk, 0)


def _k4_gather(colsf, valsf, rowsf, cnts, mvar):
    mesh = plsc.VectorSubcoreMesh(core_axis_name="c", subcore_axis_name="s")
    out_type = (
        jax.ShapeDtypeStruct((E0P, H), jnp.float32),
        jax.ShapeDtypeStruct((E0P, H), jnp.float32),
        jax.ShapeDtypeStruct((E0P,), jnp.float32),
        jax.ShapeDtypeStruct((E0P,), jnp.int32),
        jax.ShapeDtypeStruct((16,), jnp.int32),
    )
    scratch = [
        pltpu.VMEM((NW * 16,), jnp.int32),
        pltpu.VMEM((NW,), jnp.int32),
        pltpu.VMEM((1, BT), jnp.int32),
        pltpu.VMEM((1, BT), jnp.int32),
        pltpu.VMEM((1, BT), jnp.int32),
        pltpu.VMEM((BT,), jnp.float32),
        pltpu.VMEM((BT, H), jnp.float32),
        pltpu.VMEM((BT, H), jnp.float32),
        pltpu.VMEM((16,), jnp.int32),
        pltpu.SemaphoreType.DMA,
        pltpu.SemaphoreType.DMA,
        pltpu.SemaphoreType.DMA,
    ]
    return pl.kernel(_gather_body, out_type=out_type, mesh=mesh,
                     scratch_types=scratch,
                     compiler_params=pltpu.CompilerParams(
                         needs_layout_passes=False))(colsf, valsf, rowsf, cnts, mvar)


def _scatter_body(msg, rowsE, zrows, nm_o, acc, mstage, ridx):
    c = lax.axis_index("c")
    s = lax.axis_index("s")
    r0 = s * 624
    pltpu.sync_copy(zrows.at[pl.ds(r0, 624), :], acc.at[pl.ds(r0, 624), :])

    @pl.when(s == 0)
    def _():
        pltpu.sync_copy(zrows.at[pl.ds(9984, 16), :], acc.at[pl.ds(9984, 16), :])

    plsc.subcore_barrier()
    half = E0P // 2

    def chunk(t, _):
        base = c * half + s * SPW + t * BT
        pltpu.sync_copy(msg.at[pl.ds(base, BT), :], mstage)
        pltpu.sync_copy(rowsE.at[pl.ds(pl.multiple_of(base, 8), BT)], ridx.at[0])
        pltpu.sync_copy(mstage, acc.at[ridx.at[0]], add=True)
        return 0

    lax.fori_loop(0, SPW // BT, chunk, 0)
    plsc.subcore_barrier()
    pltpu.sync_copy(acc.at[pl.ds(r0, 624), :], nm_o.at[c, pl.ds(r0, 624), :])

    @pl.when(s == 0)
    def _():
        pltpu.sync_copy(acc.at[pl.ds(9984, 16), :], nm_o.at[c, pl.ds(9984, 16), :])


def _k6_scatter(msg, rowsE, zrows):
    mesh = plsc.VectorSubcoreMesh(core_axis_name="c", subcore_axis_name="s")
    out_type = jax.ShapeDtypeStruct((2, N, H), jnp.float32)
    scratch = [
        pltpu.VMEM_SHARED((N, H), jnp.float32),
        pltpu.VMEM((BT, H), jnp.float32),
        pltpu.VMEM((1, BT), jnp.int32),
    ]
    return pl.kernel(_scatter_body, out_type=out_type, mesh=mesh,
                     scratch_types=scratch,
                     compiler_params=pltpu.CompilerParams(
                         needs_layout_passes=False))(msg, rowsE, zrows)


def _stats_body(j_ref, rc_ref, rs_ref, cc_ref, cs_ref):
    x = j_ref[...]
    xb = _b16f(x)
    m = (x != 0.0).astype(jnp.float32)
    rc_ref[...] = jnp.sum(m, axis=1, keepdims=True)
    rs_ref[...] = jnp.sum(xb, axis=1, keepdims=True)

    @pl.when(pl.program_id(0) == 0)
    def _():
        cc_ref[...] = jnp.zeros_like(cc_ref)
        cs_ref[...] = jnp.zeros_like(cs_ref)

    cc_ref[...] += jnp.sum(m, axis=0, keepdims=True)
    cs_ref[...] += jnp.sum(xb, axis=0, keepdims=True)


def _k0_stats(J):
    BR = 80
    grid = (N // BR,)
    return pl.pallas_call(
        _stats_body,
        grid=grid,
        in_specs=[pl.BlockSpec((BR, N), lambda i: (i, 0))],
        out_specs=[
            pl.BlockSpec((BR, 1), lambda i: (i, 0)),
            pl.BlockSpec((BR, 1), lambda i: (i, 0)),
            pl.BlockSpec((1, N), lambda i: (0, 0)),
            pl.BlockSpec((1, N), lambda i: (0, 0)),
        ],
        out_shape=[
            jax.ShapeDtypeStruct((N, 1), jnp.float32),
            jax.ShapeDtypeStruct((N, 1), jnp.float32),
            jax.ShapeDtypeStruct((1, N), jnp.float32),
            jax.ShapeDtypeStruct((1, N), jnp.float32),
        ],
    )(J)


def _node_body(rc, rs, cc, cs, c01, c1, wih, whh, bih, bhh, mv_ref):
    deg = rc[...] + cc[...]
    sv = rs[...] + cs[...]
    c0b = _b16f(c01[...])
    agg = deg * c0b[0:1, :] + sv * c0b[1:2, :]
    gi = _dotb(agg, wih[...]) + bih[...]
    hb = bhh[...]
    r = jax.nn.sigmoid(gi[:, 0:H] + hb[:, 0:H])
    z = jax.nn.sigmoid(gi[:, H:2 * H] + hb[:, H:2 * H])
    n = jnp.tanh(gi[:, 2 * H:] + r * hb[:, 2 * H:])
    h1 = (1.0 - z) * n
    mv_ref[...] = _dotb(h1, c1[...])


def _k3_node(rc, rs, ccT, csT, c01, c1, wih, whh, bih, bhh):
    BN = 200
    grid = (N // BN,)
    full = lambda shape: pl.BlockSpec(shape, lambda i: tuple(0 for _ in shape))
    return pl.pallas_call(
        _node_body,
        grid=grid,
        in_specs=[
            pl.BlockSpec((BN, 1), lambda i: (i, 0)),
            pl.BlockSpec((BN, 1), lambda i: (i, 0)),
            pl.BlockSpec((BN, 1), lambda i: (i, 0)),
            pl.BlockSpec((BN, 1), lambda i: (i, 0)),
            full((2, H)), full((H, H)), full((H, 3 * H)), full((H, 3 * H)),
            full((1, 3 * H)), full((1, 3 * H)),
        ],
        out_specs=pl.BlockSpec((BN, H), lambda i: (i, 0)),
        out_shape=jax.ShapeDtypeStruct((N, H), jnp.float32),
    )(rc, rs, ccT, csT, c01, c1, wih, whh, bih, bhh)


def _edge_body(a1, a2, vals, nnz_ref, wih, whh, bih, bhh,
               w1, b1, w2, b2, w3, b3, msg_ref, *, TE):
    nnz = nnz_ref[0]
    vv = vals[...]
    bi = bih[...]
    bh = bhh[...]
    whh_ = whh[...]
    whb = _b16f(whh_)
    ghf = whb[0:1, :] + _b16f(vv) * whb[1:2, :] + bh
    r2 = jax.nn.sigmoid(bi[:, 0:H] + ghf[:, 0:H])
    z2 = jax.nn.sigmoid(bi[:, H:2 * H] + ghf[:, H:2 * H])
    n2 = jnp.tanh(bi[:, 2 * H:] + r2 * ghf[:, 2 * H:])
    lane = lax.broadcasted_iota(jnp.int32, (TE, H), 1)
    h0 = jnp.where(lane == 0, 1.0, 0.0) + jnp.where(lane == 1, vv, 0.0)
    h1f = (1.0 - z2) * n2 + z2 * h0
    agg = a1[...] + a2[...]
    gi2 = _dotb(agg, wih[...]) + bi
    gh2 = _dotb(h1f, whh_) + bh
    r3 = jax.nn.sigmoid(gi2[:, 0:H] + gh2[:, 0:H])
    z3 = jax.nn.sigmoid(gi2[:, H:2 * H] + gh2[:, H:2 * H])
    n3 = jnp.tanh(gi2[:, 2 * H:] + r3 * gh2[:, 2 * H:])
    h2 = (1.0 - z3) * n3 + z3 * h1f
    msg = jax.nn.relu(_dotb(h2, w1[...]) + b1[...])
    msg = jax.nn.relu(_dotb(msg, w2[...]) + b2[...])
    msg = _dotb(msg, w3[...]) + b3[...]
    eidx = pl.program_id(0) * TE + lax.broadcasted_iota(jnp.int32, (TE, 1), 0)
    msg_ref[...] = jnp.where(eidx < nnz, msg, 0.0)


def _k5_edge(a1, a2, valsE, nnz11, wih, whh, bih, bhh, w1, b1, w2, b2, w3, b3):
    TE = 512
    grid = (E0P // TE,)
    full = lambda shape: pl.BlockSpec(shape, lambda i: tuple(0 for _ in shape))
    return pl.pallas_call(
        functools.partial(_edge_body, TE=TE),
        grid=grid,
        in_specs=[
            pl.BlockSpec((TE, H), lambda i: (i, 0)),
            pl.BlockSpec((TE, H), lambda i: (i, 0)),
            pl.BlockSpec((TE, 1), lambda i: (i, 0)),
            pl.BlockSpec(memory_space=pltpu.SMEM),
            full((H, 3 * H)), full((H, 3 * H)),
            full((1, 3 * H)), full((1, 3 * H)),
            full((H, H)), full((1, H)), full((H, H)), full((1, H)),
            full((H, H)), full((1, H)),
        ],
        out_specs=pl.BlockSpec((TE, H), lambda i: (i, 0)),
        out_shape=jax.ShapeDtypeStruct((E0P, H), jnp.float32),
    )(a1, a2, valsE, nnz11, wih, whh, bih, bhh, w1, b1, w2, b2, w3, b3)


def _readout_body(nm0, nm1, w1, b1, w2, b2, w3, b3, out_ref):
    nm = nm0[...] + nm1[...]
    o = jax.nn.relu(_dotb(nm, w1[...]) + b1[...])
    o = jax.nn.relu(_dotb(o, w2[...]) + b2[...])
    o = _dotb(o, w3[...]) + b3[...]
    m = jnp.max(o, axis=1, keepdims=True)
    e = jnp.exp(o - m)
    out_ref[...] = e / jnp.sum(e, axis=1, keepdims=True)


def _k7_readout(nm0, nm1, w1, b1, w2, b2, w3, b3):
    BN = 200
    grid = (N // BN,)
    full = lambda shape: pl.BlockSpec(shape, lambda i: tuple(0 for _ in shape))
    return pl.pallas_call(
        _readout_body,
        grid=grid,
        in_specs=[
            pl.BlockSpec((BN, H), lambda i: (i, 0)),
            pl.BlockSpec((BN, H), lambda i: (i, 0)),
            full((H, H)), full((1, H)), full((H, H)), full((1, H)),
            full((H, 2)), full((1, 2)),
        ],
        out_specs=pl.BlockSpec((BN, 2), lambda i: (i, 0)),
        out_shape=jax.ShapeDtypeStruct((N, 2), jnp.float32),
    )(nm0, nm1, w1, b1, w2, b2, w3, b3)


def kernel(J, b, conv_W, gru_W_ih, gru_W_hh, gru_b_ih, gru_b_hh,
           mp_W1, mp_b1, mp_W2, mp_b2, mp_W3, mp_b3,
           ro_W1, ro_b1, ro_W2, ro_b2, ro_W3, ro_b3):
    bih = gru_b_ih.reshape(1, 3 * H)
    bhh = gru_b_hh.reshape(1, 3 * H)
    rc, rs, cc, cs = _k0_stats(J)
    mvar = _k3_node(rc, rs, cc.reshape(N, 1), cs.reshape(N, 1),
                    conv_W[0][0:2], conv_W[1], gru_W_ih, gru_W_hh, bih, bhh)
    colsF, valsF, rowsF, cnts = _k1_extract(J.reshape(-1))
    agg1, agg2, valsE, rowsE, nnzv = _k4_gather(colsF, valsF, rowsF, cnts, mvar)
    nnz11 = nnzv[0:1]
    msg = _k5_edge(agg1, agg2, valsE.reshape(E0P, 1), nnz11,
                   gru_W_ih, gru_W_hh, bih, bhh,
                   mp_W1, mp_b1.reshape(1, H), mp_W2, mp_b2.reshape(1, H),
                   mp_W3, mp_b3.reshape(1, H))
    zrows = jnp.zeros((N, H), jnp.float32)
    nm2 = _k6_scatter(msg, rowsE, zrows)
    out = _k7_readout(nm2[0], nm2[1], ro_W1, ro_b1.reshape(1, H),
                      ro_W2, ro_b2.reshape(1, H), ro_W3, ro_b3.reshape(1, 2))
    return out
